# baseline (speedup 1.0000x reference)
import jax
import jax.numpy as jnp
from jax import lax
from jax.experimental import pallas as pl
from jax.experimental.pallas import tpu as pltpu
from jax.experimental.pallas import tpu_sc as plsc

NUM_EMB = 1000000
N = 819200
D = 3
NTAB = 4

NC = 2
NS = 16
NW = NC * NS
PER_W = N // NW
CHUNK = 128
NCHUNK = PER_W // CHUNK
NPAIR = NCHUNK // 2
LANES = 16

NW_A = 25
ROWS_W = NUM_EMB // NW_A
RB = 1600
NBLK = ROWS_W // RB
GB = RB // LANES


def _band_body(w0, w1, w2, w3, band_hbm, stg, outb, sem_in, sem_out):
    cid = lax.axis_index("c")
    sid = lax.axis_index("s")
    wid = sid * NC + cid
    tables = (w0, w1, w2, w3)

    iot = lax.iota(jnp.int32, LANES)
    colv = [jnp.full((LANES,), c, jnp.int32) for c in range(D)]
    ocolv = [jnp.full((LANES,), j, jnp.int32) for j in range(NTAB * D)]

    @pl.when(wid < NW_A)
    def _():
        def block(b, carry):
            row0 = wid * ROWS_W + b * RB
            hs = [pltpu.async_copy(tables[t].at[pl.ds(row0, RB)],
                                   stg.at[pl.ds(t * RB, RB)], sem_in)
                  for t in range(NTAB)]
            for h in hs:
                h.wait()

            def grp(g, c2):
                r0 = g * LANES
                rows_out = iot + r0
                for t in range(NTAB):
                    rows_in = iot + (t * RB + r0)
                    for c in range(D):
                        strip = plsc.load_gather(stg, [rows_in, colv[c]])
                        plsc.store_scatter(outb, [rows_out, ocolv[t * D + c]],
                                           strip)
                return c2

            lax.fori_loop(0, GB, grp, 0)
            pltpu.sync_copy(outb, band_hbm.at[pl.ds(row0, RB)])
            return carry

        lax.fori_loop(0, NBLK, block, 0)


def _gather_body(idx_hbm, tab_hbm, zacc_hbm, out_hbm,
                 idx_v, buf_a, buf_b, acc, sidx_v, sem_a, sem_b):
    cid = lax.axis_index("c")
    sid = lax.axis_index("s")
    wid = sid * NC + cid
    pltpu.sync_copy(idx_hbm.at[wid], idx_v)

    sid_vec = jnp.full((LANES,), sid, jnp.int32)
    for k in range(CHUNK // LANES):
        sidx_v[pl.ds(k * LANES, LANES)] = sid_vec

    @pl.when(sid == 0)
    def _():
        pltpu.sync_copy(zacc_hbm, acc)

    plsc.subcore_barrier()

    def body(g, carry):
        c0 = 2 * g
        h_a = pltpu.async_copy(tab_hbm.at[idx_v.at[c0]], buf_a, sem_a)
        h_b = pltpu.async_copy(tab_hbm.at[idx_v.at[c0 + 1]], buf_b, sem_b)
        h_a.wait()
        pltpu.sync_copy(buf_a, acc.at[sidx_v], add=True)
        h_b.wait()
        pltpu.sync_copy(buf_b, acc.at[sidx_v], add=True)
        return carry

    lax.fori_loop(0, NPAIR, body, 0)
    plsc.subcore_barrier()

    @pl.when(sid == 0)
    def _():
        pltpu.sync_copy(acc, out_hbm.at[cid])


def kernel(eb_input, eb_offset, W0, W1, W2, W3):
    del eb_offset
    idx3 = eb_input.reshape(NW, NCHUNK, CHUNK)
    zacc = jnp.zeros((NS, LANES), jnp.float32)

    mesh = plsc.VectorSubcoreMesh(core_axis_name="c", subcore_axis_name="s")
    cparams = pltpu.CompilerParams(needs_layout_passes=False,
                                   use_tc_tiling_on_sc=False)

    run_band = pl.kernel(
        _band_body,
        out_type=jax.ShapeDtypeStruct((NUM_EMB, LANES), jnp.float32),
        mesh=mesh,
        scratch_types=[
            pltpu.VMEM((NTAB * RB, D), jnp.float32),
            pltpu.VMEM((RB, LANES), jnp.float32),
            pltpu.SemaphoreType.DMA,
            pltpu.SemaphoreType.DMA,
        ],
        compiler_params=cparams,
    )

    run_gather = pl.kernel(
        _gather_body,
        out_type=jax.ShapeDtypeStruct((NC, NS, LANES), jnp.float32),
        mesh=mesh,
        scratch_types=[
            pltpu.VMEM((NCHUNK, CHUNK), jnp.int32),
            pltpu.VMEM((CHUNK, LANES), jnp.float32),
            pltpu.VMEM((CHUNK, LANES), jnp.float32),
            pltpu.VMEM_SHARED((NS, LANES), jnp.float32),
            pltpu.VMEM((CHUNK,), jnp.int32),
            pltpu.SemaphoreType.DMA,
            pltpu.SemaphoreType.DMA,
        ],
        compiler_params=cparams,
    )

    band = run_band(W0, W1, W2, W3)
    partials = run_gather(idx3, band, zacc)
    lanes = partials.sum(axis=(0, 1))
    return lanes[:NTAB * D].reshape(NTAB, D).sum(axis=0)


# speedup vs baseline: 1.0878x; 1.0878x over previous
import jax
import jax.numpy as jnp
from jax import lax
from jax.experimental import pallas as pl
from jax.experimental.pallas import tpu as pltpu
from jax.experimental.pallas import tpu_sc as plsc

NUM_EMB = 1000000
N = 819200
D = 3
NTAB = 4

NC = 2
NS = 16
NW = NC * NS
PER_W = N // NW
CHUNK = 128
NCHUNK = PER_W // CHUNK
NPAIR = NCHUNK // 2
LANES = 16

NW_A = 25
ROWS_W = NUM_EMB // NW_A
RB = 1600
NBLK = ROWS_W // RB
GB = RB // LANES


def _band_body(w0, w1, w2, w3, band_hbm, stg, outb, sem_in, sem_out):
    cid = lax.axis_index("c")
    sid = lax.axis_index("s")
    wid = sid * NC + cid
    tables = (w0, w1, w2, w3)

    iot = lax.iota(jnp.int32, LANES)
    iot3 = iot * D
    tfull = [jnp.full((LANES,), t, jnp.int32) for t in range(NTAB)]
    ocolv = [jnp.full((LANES,), j, jnp.int32) for j in range(NTAB * D)]

    @pl.when(wid < NW_A)
    def _():
        def block(b, carry):
            row0 = wid * ROWS_W + b * RB
            hs = [pltpu.async_copy(tables[t].at[pl.ds(row0 * D, RB * D)],
                                   stg.at[t], sem_in)
                  for t in range(NTAB)]
            for h in hs:
                h.wait()

            def grp(g, c2):
                r0 = g * LANES
                rows_out = iot + r0
                for t in range(NTAB):
                    for c in range(D):
                        words = iot3 + (D * r0 + c)
                        strip = plsc.load_gather(stg, [tfull[t], words])
                        plsc.store_scatter(outb, [rows_out, ocolv[t * D + c]],
                                           strip)
                return c2

            lax.fori_loop(0, GB, grp, 0)
            pltpu.sync_copy(outb, band_hbm.at[pl.ds(row0, RB)])
            return carry

        lax.fori_loop(0, NBLK, block, 0)


def _gather_body(idx_hbm, tab_hbm, zacc_hbm, out_hbm,
                 idx_v, buf_a, buf_b, acc, sidx_v, sem_a, sem_b):
    cid = lax.axis_index("c")
    sid = lax.axis_index("s")
    wid = sid * NC + cid
    pltpu.sync_copy(idx_hbm.at[wid], idx_v)

    sid_vec = jnp.full((LANES,), sid, jnp.int32)
    for k in range(CHUNK // LANES):
        sidx_v[pl.ds(k * LANES, LANES)] = sid_vec

    @pl.when(sid == 0)
    def _():
        pltpu.sync_copy(zacc_hbm, acc)

    plsc.subcore_barrier()

    def body(g, carry):
        c0 = 2 * g
        h_a = pltpu.async_copy(tab_hbm.at[idx_v.at[c0]], buf_a, sem_a)
        h_b = pltpu.async_copy(tab_hbm.at[idx_v.at[c0 + 1]], buf_b, sem_b)
        h_a.wait()
        pltpu.sync_copy(buf_a, acc.at[sidx_v], add=True)
        h_b.wait()
        pltpu.sync_copy(buf_b, acc.at[sidx_v], add=True)
        return carry

    lax.fori_loop(0, NPAIR, body, 0)
    plsc.subcore_barrier()

    @pl.when(sid == 0)
    def _():
        pltpu.sync_copy(acc, out_hbm.at[cid])


def kernel(eb_input, eb_offset, W0, W1, W2, W3):
    del eb_offset
    idx3 = eb_input.reshape(NW, NCHUNK, CHUNK)
    zacc = jnp.zeros((NS, LANES), jnp.float32)

    mesh = plsc.VectorSubcoreMesh(core_axis_name="c", subcore_axis_name="s")
    cparams = pltpu.CompilerParams(needs_layout_passes=False,
                                   use_tc_tiling_on_sc=False)

    run_band = pl.kernel(
        _band_body,
        out_type=jax.ShapeDtypeStruct((NUM_EMB, LANES), jnp.float32),
        mesh=mesh,
        scratch_types=[
            pltpu.VMEM((NTAB, RB * D), jnp.float32),
            pltpu.VMEM((RB, LANES), jnp.float32),
            pltpu.SemaphoreType.DMA,
            pltpu.SemaphoreType.DMA,
        ],
        compiler_params=cparams,
    )

    run_gather = pl.kernel(
        _gather_body,
        out_type=jax.ShapeDtypeStruct((NC, NS, LANES), jnp.float32),
        mesh=mesh,
        scratch_types=[
            pltpu.VMEM((NCHUNK, CHUNK), jnp.int32),
            pltpu.VMEM((CHUNK, LANES), jnp.float32),
            pltpu.VMEM((CHUNK, LANES), jnp.float32),
            pltpu.VMEM_SHARED((NS, LANES), jnp.float32),
            pltpu.VMEM((CHUNK,), jnp.int32),
            pltpu.SemaphoreType.DMA,
            pltpu.SemaphoreType.DMA,
        ],
        compiler_params=cparams,
    )

    band = run_band(W0.reshape(-1), W1.reshape(-1), W2.reshape(-1),
                    W3.reshape(-1))
    partials = run_gather(idx3, band, zacc)
    lanes = partials.sum(axis=(0, 1))
    return lanes[:NTAB * D].reshape(NTAB, D).sum(axis=0)
